# baseline (speedup 1.0000x reference)
import dataclasses

import jax
import jax.numpy as jnp
from jax import lax
from jax.experimental import pallas as pl
from jax.experimental.pallas import tpu as pltpu
from jax.experimental.pallas import tpu_sc as plsc

B = 16384
S = 3
K = 4
SK = S * K
V = 1000000
NC, NS = 2, 16
NW = NC * NS
BPW = B // NW
EPW = BPW * SK
PPW = BPW * S
CHUNK = 128
NCHUNK = BPW // CHUNK
GA_ROWS = V * SK // 16
GB_ROWS = V * S // 16


def _cos(x):
    inv_2pi = jnp.float32(0.15915494309189535)
    two_pi = jnp.float32(6.283185307179586)
    t = x * inv_2pi
    half = jnp.where(t >= 0, jnp.float32(0.5), jnp.float32(-0.5))
    n = (t + half).astype(jnp.int32).astype(jnp.float32)
    r = x - n * two_pi
    r2 = r * r
    p = jnp.float32(-1.1470745597729725e-11)
    p = p * r2 + jnp.float32(2.08767569878681e-09)
    p = p * r2 + jnp.float32(-2.755731922398589e-07)
    p = p * r2 + jnp.float32(2.48015873015873e-05)
    p = p * r2 + jnp.float32(-0.001388888888888889)
    p = p * r2 + jnp.float32(0.041666666666666664)
    p = p * r2 + jnp.float32(-0.5)
    return p * r2 + jnp.float32(1.0)


def _body(amp_hbm, phase_hbm, bias_hbm, ids_hbm, sa_hbm, sp_hbm, gs_hbm,
          na_hbm, np_hbm,
          ids_v, idx_a0, idx_a1, idx_b0, idx_b1, offa_v, offb_v,
          blk_a0, blk_a1, blk_p0, blk_p1, blk_b0, blk_b1,
          sa_v, sp_v, da_v, dp_v, c_v, gate_v, gs_v, sem_g, sem_s):
    wid = lax.axis_index("c") * NS + lax.axis_index("s")
    el0 = wid * EPW

    pltpu.sync_copy(ids_hbm.at[pl.ds(wid * BPW, BPW)], ids_v)
    st_copies = [
        pltpu.async_copy(sa_hbm.at[pl.ds(el0, EPW)], sa_v, sem_s),
        pltpu.async_copy(sp_hbm.at[pl.ds(el0, EPW)], sp_v, sem_s),
    ]
    pltpu.sync_copy(gs_hbm, gs_v)

    for it in range(BPW // 16):
        j, c = it // (CHUNK // 16), 16 * (it % (CHUNK // 16))
        sl = pl.ds(16 * it, 16)
        idv = ids_v[sl]
        i3 = idv * 3
        ra0 = lax.shift_right_logical(i3, 2)
        rb0 = lax.shift_right_logical(i3, 4)
        csl = pl.ds(c, 16)
        idx_a0[j, csl] = ra0
        idx_a1[j, csl] = jnp.minimum(ra0 + 1, GA_ROWS - 1)
        idx_b0[j, csl] = rb0
        idx_b1[j, csl] = jnp.minimum(rb0 + 1, GB_ROWS - 1)
        offa_v[sl] = lax.shift_left(i3 & 3, 2)
        offb_v[sl] = i3 & 15

    copies = []
    for j in range(NCHUNK):
        dst = pl.ds(j * CHUNK, CHUNK)
        for tab, ix, blk in ((amp_hbm, idx_a0, blk_a0),
                             (amp_hbm, idx_a1, blk_a1),
                             (phase_hbm, idx_a0, blk_p0),
                             (phase_hbm, idx_a1, blk_p1),
                             (bias_hbm, idx_b0, blk_b0),
                             (bias_hbm, idx_b1, blk_b1)):
            copies.append(pltpu.async_copy(tab.at[ix.at[j]], blk.at[dst],
                                           sem_g))
    for c in copies:
        c.wait()
    for c in st_copies:
        c.wait()

    lane = lax.iota(jnp.int32, 16)
    ra = [(lane + 16 * m) // SK for m in range(3)]
    ca = [(lane + 16 * m) % SK for m in range(3)]

    @pl.loop(0, BPW // 4)
    def _pass_a(j):
        for m in range(3):
            base = 48 * j + 16 * m
            t = ra[m] + 4 * j
            off = plsc.load_gather(offa_v, [t])
            cc = off + ca[m]
            col = cc & 15
            lo = cc < 16
            da = jnp.where(lo, plsc.load_gather(blk_a0, [t, col]),
                           plsc.load_gather(blk_a1, [t, col]))
            dp = jnp.where(lo, plsc.load_gather(blk_p0, [t, col]),
                           plsc.load_gather(blk_p1, [t, col]))
            sl = pl.ds(base, 16)
            c_v[sl] = sa_v[sl] * da * _cos(sp_v[sl] - dp)
            da_v[sl] = da
            dp_v[sl] = dp

    rb = [(lane + 16 * m) // S for m in range(3)]
    cb = [(lane + 16 * m) % S for m in range(3)]
    gs = gs_v[pl.ds(0, 16)]

    @pl.loop(0, PPW // 48)
    def _pass_b(i):
        for m in range(3):
            q0 = 48 * i + 16 * m
            e0 = (q0 + lane) * 4
            acc = plsc.load_gather(c_v, [e0])
            for k in range(1, K):
                acc = acc + plsc.load_gather(c_v, [e0 + k])
            coh = acc * jnp.float32(1.0 / K)
            t = rb[m] + 16 * i
            off = plsc.load_gather(offb_v, [t])
            cc = off + cb[m]
            col = cc & 15
            gb = jnp.where(cc < 16, plsc.load_gather(blk_b0, [t, col]),
                           plsc.load_gather(blk_b1, [t, col]))
            logit = gb + gs * coh
            gate_v[pl.ds(q0, 16)] = jnp.float32(1.0) / (
                jnp.float32(1.0) + jnp.exp(-logit))

    @pl.loop(0, BPW // 4)
    def _pass_c(j):
        for m in range(3):
            base = 48 * j + 16 * m
            sl = pl.ds(base, 16)
            e = base + lane
            g = plsc.load_gather(gate_v, [lax.shift_right_logical(e, 2)])
            one_m_g = jnp.float32(1.0) - g
            sa_v[sl] = g * sa_v[sl] + one_m_g * da_v[sl]
            sp_v[sl] = g * sp_v[sl] + one_m_g * dp_v[sl]

    out_sl = pl.ds(el0, EPW)
    pltpu.sync_copy(sa_v, na_hbm.at[out_sl])
    pltpu.sync_copy(sp_v, np_hbm.at[out_sl])


def _compiler_params():
    cp = pltpu.CompilerParams()
    fields = pltpu.CompilerParams.__dataclass_fields__
    if "needs_layout_passes" in fields:
        cp = dataclasses.replace(cp, needs_layout_passes=False)
    if "use_tc_tiling_on_sc" in fields:
        cp = dataclasses.replace(cp, use_tc_tiling_on_sc=False)
    return cp


@jax.jit
def _gse(ids1d, amp16, phase16, bias16, sa_flat, sp_flat, gs16):
    f32 = jnp.float32
    i32 = jnp.int32
    run = pl.kernel(
        _body,
        compiler_params=_compiler_params(),
        out_type=(jax.ShapeDtypeStruct((B * SK,), f32),
                  jax.ShapeDtypeStruct((B * SK,), f32)),
        mesh=plsc.VectorSubcoreMesh(core_axis_name="c", subcore_axis_name="s"),
        scratch_types=[
            pltpu.VMEM((BPW,), i32),
            pltpu.VMEM((NCHUNK, CHUNK), i32),
            pltpu.VMEM((NCHUNK, CHUNK), i32),
            pltpu.VMEM((NCHUNK, CHUNK), i32),
            pltpu.VMEM((NCHUNK, CHUNK), i32),
            pltpu.VMEM((BPW,), i32),
            pltpu.VMEM((BPW,), i32),
            pltpu.VMEM((BPW, 16), f32),
            pltpu.VMEM((BPW, 16), f32),
            pltpu.VMEM((BPW, 16), f32),
            pltpu.VMEM((BPW, 16), f32),
            pltpu.VMEM((BPW, 16), f32),
            pltpu.VMEM((BPW, 16), f32),
            pltpu.VMEM((EPW,), f32),
            pltpu.VMEM((EPW,), f32),
            pltpu.VMEM((EPW,), f32),
            pltpu.VMEM((EPW,), f32),
            pltpu.VMEM((EPW,), f32),
            pltpu.VMEM((PPW,), f32),
            pltpu.VMEM((16,), f32),
            pltpu.SemaphoreType.DMA,
            pltpu.SemaphoreType.DMA,
        ],
    )
    return run(amp16, phase16, bias16, ids1d, sa_flat, sp_flat, gs16)


def kernel(ids, state_amp, state_phase, delta_amp, delta_phase, gate_bias,
           gate_sensitivity):
    ids1d = ids.astype(jnp.int32).reshape(B)
    amp16 = delta_amp.reshape(GA_ROWS, 16)
    phase16 = delta_phase.reshape(GA_ROWS, 16)
    bias16 = gate_bias.reshape(GB_ROWS, 16)
    sa_flat = state_amp.reshape(B * SK)
    sp_flat = state_phase.reshape(B * SK)
    gs16 = jnp.broadcast_to(
        gate_sensitivity.astype(jnp.float32).reshape(()), (16,))
    na, nph = _gse(ids1d, amp16, phase16, bias16, sa_flat, sp_flat, gs16)
    return na.reshape(B, S, K), nph.reshape(B, S, K)


# speedup vs baseline: 18.1231x; 18.1231x over previous
import dataclasses

import jax
import jax.numpy as jnp
from jax import lax
from jax.experimental import pallas as pl
from jax.experimental.pallas import tpu as pltpu
from jax.experimental.pallas import tpu_sc as plsc

B = 16384
S = 3
K = 4
V = 1000000
NC, NS = 2, 16
NW = NC * NS
BPW = B // NW
TCH = 128
NCH = BPW // TCH

G = 32
NT = -(-V // 128)
NB = -(-NT // G)
NTP = NB * G
SPL = NTP * 512
SGR = SPL // 16
DROWS = 3 * NTP * 4
BROWS = NTP * 3

NTS = B // 128
NBS = NTS // G
SROWS = 3 * NTS * 4


def _cos(x):
    t = x * jnp.float32(0.15915494309189535)
    half = jnp.where(t >= 0, jnp.float32(0.5), jnp.float32(-0.5))
    n = (t + half).astype(jnp.int32).astype(jnp.float32)
    r = x - n * jnp.float32(6.283185307179586)
    r2 = r * r
    p = jnp.float32(-1.1470745597729725e-11)
    p = p * r2 + jnp.float32(2.08767569878681e-09)
    p = p * r2 + jnp.float32(-2.755731922398589e-07)
    p = p * r2 + jnp.float32(2.48015873015873e-05)
    p = p * r2 + jnp.float32(-0.001388888888888889)
    p = p * r2 + jnp.float32(0.041666666666666664)
    p = p * r2 + jnp.float32(-0.5)
    return p * r2 + jnp.float32(1.0)


def _detile_body(t_ref, o_ref):
    x = t_ref[...]
    sd = x.shape[1]
    parts = [x[0, :, 128 * g:128 * (g + 1)] for g in range(x.shape[2] // 128)]
    o_ref[...] = jnp.concatenate(parts, axis=0)


def _detile2_body(a_ref, b_ref, oa_ref, ob_ref):
    for src, dst in ((a_ref, oa_ref), (b_ref, ob_ref)):
        x = src[...]
        parts = [x[0, :, 128 * g:128 * (g + 1)]
                 for g in range(x.shape[2] // 128)]
        dst[...] = jnp.concatenate(parts, axis=0)


def _detile_pair(a, b, sd, n_sblk, g, nblk):
    rpb = sd * g
    spec_i = pl.BlockSpec((1, sd, 128 * g), lambda s, i: (s, 0, i))
    spec_o = pl.BlockSpec((rpb, 128), lambda s, i: (s * nblk + i, 0))
    rows = n_sblk * nblk * rpb
    return pl.pallas_call(
        _detile2_body,
        grid=(n_sblk, nblk),
        in_specs=[spec_i, spec_i],
        out_specs=[spec_o, spec_o],
        out_shape=[jax.ShapeDtypeStruct((rows, 128), jnp.float32)] * 2,
    )(a, b)


def _detile_one(a, sd, n_sblk, g, nblk):
    rpb = sd * g
    rows = n_sblk * nblk * rpb
    return pl.pallas_call(
        _detile_body,
        grid=(n_sblk, nblk),
        in_specs=[pl.BlockSpec((1, sd, 128 * g), lambda s, i: (s, 0, i))],
        out_specs=pl.BlockSpec((rpb, 128), lambda s, i: (s * nblk + i, 0)),
        out_shape=jax.ShapeDtypeStruct((rows, 128), jnp.float32),
    )(a)


def _sc_body(ampg, phaseg, biasg, ids_hbm, sa_hbm, sp_hbm, gs_hbm,
             na_hbm, np_hbm,
             ids_v, offs_v, idx3, blk_a, blk_p, blk_b,
             sa_v, sp_v, cs_v, daf_v, dpf_v, gate_v, gs_v, sem_g, sem_s):
    wid = lax.axis_index("c") * NS + lax.axis_index("s")
    tok0 = wid * BPW

    pltpu.sync_copy(ids_hbm.at[pl.ds(tok0, BPW)], ids_v)
    st = []
    for s in range(S):
        src = pl.ds(s * (B * K) + wid * BPW * K, BPW * K)
        dst = pl.ds(s * BPW * K, BPW * K)
        st.append(pltpu.async_copy(sa_hbm.at[src], sa_v.at[dst], sem_s))
        st.append(pltpu.async_copy(sp_hbm.at[src], sp_v.at[dst], sem_s))
    pltpu.sync_copy(gs_hbm, gs_v)

    lane = lax.iota(jnp.int32, 16)

    @pl.loop(0, BPW // 16)
    def _mkidx(u):
        c = lax.shift_right_logical(u, 3)
        lg = u & 7
        idv = ids_v[pl.ds(16 * u, 16)]
        vt = lax.shift_right_logical(idv, 7)
        sub = lax.shift_right_logical(idv, 4) & 7
        base_a = vt * 32 + sub
        base_b = vt * 24 + sub
        offs_v[pl.ds(16 * u, 16)] = idv & 15
        sl = pl.ds(16 * lg, 16)
        for p in range(12):
            idx3[c, p, sl] = base_a + ((p // 4) * SGR + (p % 4) * 8)
        for s in range(S):
            idx3[c, 12 + s, sl] = base_b + s * 8

    gs = gs_v[pl.ds(0, 16)]
    for cp in st:
        cp.wait()

    @pl.loop(0, NCH)
    def _chunk(c):
        @pl.loop(0, 12)
        def _fire_d(p):
            pltpu.async_copy(ampg.at[idx3.at[c, p]], blk_a.at[p], sem_g)
            pltpu.async_copy(phaseg.at[idx3.at[c, p]], blk_p.at[p], sem_g)

        @pl.loop(0, S)
        def _fire_b(s):
            pltpu.async_copy(biasg.at[idx3.at[c, 12 + s]], blk_b.at[s], sem_g)

        @pl.loop(0, 27)
        def _drain(i):
            pltpu.make_async_copy(
                ampg.at[pl.ds(0, TCH)], blk_a.at[0], sem_g).wait()

        @pl.loop(0, 12 * (TCH // 16))
        def _pa(i):
            p = lax.shift_right_logical(i, 3)
            u = i & 7
            s = lax.shift_right_logical(p, 2)
            k = p & 3
            st_base = s * (BPW * K) + c * 512 + k * 128
            t = 16 * u + lane
            off = offs_v[pl.ds(c * TCH + 16 * u, 16)]
            pvec = jnp.full((16,), 0, jnp.int32) + p
            da = plsc.load_gather(blk_a, [pvec, t, off])
            dp = plsc.load_gather(blk_p, [pvec, t, off])
            ssl = pl.ds(st_base + 16 * u, 16)
            csl = pl.ds(p * TCH + 16 * u, 16)
            cc = sa_v[ssl] * da * _cos(sp_v[ssl] - dp)
            cs_v[csl] = cc
            daf_v[csl] = da
            dpf_v[csl] = dp

        @pl.loop(0, S * (TCH // 16))
        def _pb(i):
            s = lax.shift_right_logical(i, 3)
            u = i & 7
            t = 16 * u + lane
            acc = cs_v[pl.ds(s * 4 * TCH + 16 * u, 16)]
            for k in range(1, K):
                acc = acc + cs_v[pl.ds((s * 4 + k) * TCH + 16 * u, 16)]
            off = offs_v[pl.ds(c * TCH + 16 * u, 16)]
            svec = jnp.full((16,), 0, jnp.int32) + s
            gb = plsc.load_gather(blk_b, [svec, t, off])
            logit = gb + gs * (acc * jnp.float32(0.25))
            gate_v[pl.ds(s * TCH + 16 * u, 16)] = jnp.float32(1.0) / (
                jnp.float32(1.0) + jnp.exp(-logit))

        @pl.loop(0, 12 * (TCH // 16))
        def _pc(i):
            p = lax.shift_right_logical(i, 3)
            u = i & 7
            s = lax.shift_right_logical(p, 2)
            k = p & 3
            st_base = s * (BPW * K) + c * 512 + k * 128
            gsl = pl.ds(s * TCH + 16 * u, 16)
            ssl = pl.ds(st_base + 16 * u, 16)
            csl = pl.ds(p * TCH + 16 * u, 16)
            g = gate_v[gsl]
            omg = jnp.float32(1.0) - g
            sa_v[ssl] = g * sa_v[ssl] + omg * daf_v[csl]
            sp_v[ssl] = g * sp_v[ssl] + omg * dpf_v[csl]

    for s in range(S):
        src = pl.ds(s * BPW * K, BPW * K)
        dst = pl.ds(s * (B * K) + wid * BPW * K, BPW * K)
        pltpu.sync_copy(sa_v.at[src], na_hbm.at[dst])
        pltpu.sync_copy(sp_v.at[src], np_hbm.at[dst])


def _compiler_params():
    cp = pltpu.CompilerParams()
    fields = pltpu.CompilerParams.__dataclass_fields__
    if "needs_layout_passes" in fields:
        cp = dataclasses.replace(cp, needs_layout_passes=False)
    if "use_tc_tiling_on_sc" in fields:
        cp = dataclasses.replace(cp, use_tc_tiling_on_sc=False)
    return cp


def _sc_call(ampg, phaseg, biasg, ids1d, sa_lin, sp_lin, gs16):
    f32 = jnp.float32
    i32 = jnp.int32
    run = pl.kernel(
        _sc_body,
        compiler_params=_compiler_params(),
        out_type=(jax.ShapeDtypeStruct((B * S * K,), f32),
                  jax.ShapeDtypeStruct((B * S * K,), f32)),
        mesh=plsc.VectorSubcoreMesh(core_axis_name="c", subcore_axis_name="s"),
        scratch_types=[
            pltpu.VMEM((BPW,), i32),
            pltpu.VMEM((BPW,), i32),
            pltpu.VMEM((NCH, 15, TCH), i32),
            pltpu.VMEM((12, TCH, 16), f32),
            pltpu.VMEM((12, TCH, 16), f32),
            pltpu.VMEM((S, TCH, 16), f32),
            pltpu.VMEM((BPW * S * K,), f32),
            pltpu.VMEM((BPW * S * K,), f32),
            pltpu.VMEM((12 * TCH,), f32),
            pltpu.VMEM((12 * TCH,), f32),
            pltpu.VMEM((12 * TCH,), f32),
            pltpu.VMEM((S * TCH,), f32),
            pltpu.VMEM((16,), f32),
            pltpu.SemaphoreType.DMA,
            pltpu.SemaphoreType.DMA,
        ],
    )
    return run(ampg, phaseg, biasg, ids1d, sa_lin, sp_lin, gs16)


@jax.jit
def _gse(ids, state_amp, state_phase, delta_amp, delta_phase, gate_bias,
         gate_sensitivity):
    t_amp = jnp.transpose(delta_amp, (1, 2, 0))
    t_phase = jnp.transpose(delta_phase, (1, 2, 0))
    t_bias = jnp.transpose(gate_bias, (1, 0))[None]
    t_sa = jnp.transpose(state_amp, (1, 2, 0))
    t_sp = jnp.transpose(state_phase, (1, 2, 0))

    amp_lin, phase_lin = _detile_pair(t_amp, t_phase, K, 3, G, NB)
    bias_lin = _detile_one(t_bias, S, 1, G, NB)
    sa_lin, sp_lin = _detile_pair(t_sa, t_sp, K, 3, G, NBS)

    ampg = amp_lin.reshape(DROWS * 8, 16)
    phaseg = phase_lin.reshape(DROWS * 8, 16)
    biasg = bias_lin.reshape(BROWS * 8, 16)
    ids1d = ids.astype(jnp.int32).reshape(B)
    gs16 = jnp.broadcast_to(
        gate_sensitivity.astype(jnp.float32).reshape(()), (16,))

    na_lin, np_lin = _sc_call(ampg, phaseg, biasg, ids1d,
                              sa_lin.reshape(B * S * K),
                              sp_lin.reshape(B * S * K), gs16)

    def back(x):
        x = x.reshape(S, NTS, K, 128)
        x = jnp.transpose(x, (1, 3, 0, 2))
        return x.reshape(B, S, K)

    return back(na_lin), back(np_lin)


def kernel(ids, state_amp, state_phase, delta_amp, delta_phase, gate_bias,
           gate_sensitivity):
    return _gse(ids, state_amp, state_phase, delta_amp, delta_phase,
                gate_bias, gate_sensitivity)


# speedup vs baseline: 62.4729x; 3.4471x over previous
import dataclasses

import jax
import jax.numpy as jnp
from jax import lax
from jax.experimental import pallas as pl
from jax.experimental.pallas import tpu as pltpu
from jax.experimental.pallas import tpu_sc as plsc

B = 16384
S = 3
K = 4
V = 1000000
NC, NS = 2, 16
NW = NC * NS
BPW = B // NW
TCH = 128
NCH = BPW // TCH

G = 512
GB = 128
GS = 128
NT = -(-V // 128)
NB = -(-NT // G)
NTP = NB * G
NBB = -(-NT // GB)
NTPB = NBB * GB
SPL = NTP * 512
SGR = SPL // 16
DROWS = 3 * NTP * 4
BROWS = NTPB * 3

NTS = B // 128
NBS = NTS // GS
SROWS = 3 * NTS * 4


def _cos(x):
    t = x * jnp.float32(0.15915494309189535)
    half = jnp.where(t >= 0, jnp.float32(0.5), jnp.float32(-0.5))
    n = (t + half).astype(jnp.int32).astype(jnp.float32)
    r = x - n * jnp.float32(6.283185307179586)
    r2 = r * r
    p = jnp.float32(-1.1470745597729725e-11)
    p = p * r2 + jnp.float32(2.08767569878681e-09)
    p = p * r2 + jnp.float32(-2.755731922398589e-07)
    p = p * r2 + jnp.float32(2.48015873015873e-05)
    p = p * r2 + jnp.float32(-0.001388888888888889)
    p = p * r2 + jnp.float32(0.041666666666666664)
    p = p * r2 + jnp.float32(-0.5)
    return p * r2 + jnp.float32(1.0)


def _detile_body(t_ref, o_ref):
    x = t_ref[...]
    sd = x.shape[1]
    parts = [x[0, :, 128 * g:128 * (g + 1)] for g in range(x.shape[2] // 128)]
    o_ref[...] = jnp.concatenate(parts, axis=0)


def _detile2_body(a_ref, b_ref, oa_ref, ob_ref):
    for src, dst in ((a_ref, oa_ref), (b_ref, ob_ref)):
        x = src[...]
        parts = [x[0, :, 128 * g:128 * (g + 1)]
                 for g in range(x.shape[2] // 128)]
        dst[...] = jnp.concatenate(parts, axis=0)


def _detile_pair(a, b, sd, n_sblk, g, nblk):
    rpb = sd * g
    spec_i = pl.BlockSpec((1, sd, 128 * g), lambda s, i: (s, 0, i))
    spec_o = pl.BlockSpec((rpb, 128), lambda s, i: (s * nblk + i, 0))
    rows = n_sblk * nblk * rpb
    return pl.pallas_call(
        _detile2_body,
        grid=(n_sblk, nblk),
        in_specs=[spec_i, spec_i],
        out_specs=[spec_o, spec_o],
        out_shape=[jax.ShapeDtypeStruct((rows, 128), jnp.float32)] * 2,
    )(a, b)


def _detile_one(a, sd, n_sblk, g, nblk):
    rpb = sd * g
    rows = n_sblk * nblk * rpb
    return pl.pallas_call(
        _detile_body,
        grid=(n_sblk, nblk),
        in_specs=[pl.BlockSpec((1, sd, 128 * g), lambda s, i: (s, 0, i))],
        out_specs=pl.BlockSpec((rpb, 128), lambda s, i: (s * nblk + i, 0)),
        out_shape=jax.ShapeDtypeStruct((rows, 128), jnp.float32),
    )(a)


def _sc_body(ampg, phaseg, biasg, ids_hbm, sa_hbm, sp_hbm, gs_hbm,
             na_hbm, np_hbm,
             ids_v, offs_v, idx3, blk_a, blk_p, blk_b,
             sa_v, sp_v, cs_v, daf_v, dpf_v, gate_v, gs_v, sem_g, sem_s):
    wid = lax.axis_index("c") * NS + lax.axis_index("s")
    tok0 = wid * BPW

    pltpu.sync_copy(ids_hbm.at[pl.ds(tok0, BPW)], ids_v)
    st = []
    for s in range(S):
        src = pl.ds(s * (B * K) + wid * BPW * K, BPW * K)
        dst = pl.ds(s * BPW * K, BPW * K)
        st.append(pltpu.async_copy(sa_hbm.at[src], sa_v.at[dst], sem_s))
        st.append(pltpu.async_copy(sp_hbm.at[src], sp_v.at[dst], sem_s))
    pltpu.sync_copy(gs_hbm, gs_v)

    lane = lax.iota(jnp.int32, 16)

    @pl.loop(0, BPW // 16)
    def _mkidx(u):
        c = lax.shift_right_logical(u, 3)
        lg = u & 7
        idv = ids_v[pl.ds(16 * u, 16)]
        vt = lax.shift_right_logical(idv, 7)
        sub = lax.shift_right_logical(idv, 4) & 7
        base_a = vt * 32 + sub
        base_b = vt * 24 + sub
        offs_v[pl.ds(16 * u, 16)] = idv & 15
        sl = pl.ds(16 * lg, 16)
        for p in range(12):
            idx3[c, p, sl] = base_a + ((p // 4) * SGR + (p % 4) * 8)
        for s in range(S):
            idx3[c, 12 + s, sl] = base_b + s * 8

    gs = gs_v[pl.ds(0, 16)]
    for cp in st:
        cp.wait()

    @pl.loop(0, NCH)
    def _chunk(c):
        @pl.loop(0, 12)
        def _fire_d(p):
            pltpu.async_copy(ampg.at[idx3.at[c, p]], blk_a.at[p], sem_g)
            pltpu.async_copy(phaseg.at[idx3.at[c, p]], blk_p.at[p], sem_g)

        @pl.loop(0, S)
        def _fire_b(s):
            pltpu.async_copy(biasg.at[idx3.at[c, 12 + s]], blk_b.at[s], sem_g)

        @pl.loop(0, 27)
        def _drain(i):
            pltpu.make_async_copy(
                ampg.at[pl.ds(0, TCH)], blk_a.at[0], sem_g).wait()

        @pl.loop(0, 12 * (TCH // 16))
        def _pa(i):
            p = lax.shift_right_logical(i, 3)
            u = i & 7
            s = lax.shift_right_logical(p, 2)
            k = p & 3
            st_base = s * (BPW * K) + c * 512 + k * 128
            t = 16 * u + lane
            off = offs_v[pl.ds(c * TCH + 16 * u, 16)]
            pvec = jnp.full((16,), 0, jnp.int32) + p
            da = plsc.load_gather(blk_a, [pvec, t, off])
            dp = plsc.load_gather(blk_p, [pvec, t, off])
            ssl = pl.ds(st_base + 16 * u, 16)
            csl = pl.ds(p * TCH + 16 * u, 16)
            cc = sa_v[ssl] * da * _cos(sp_v[ssl] - dp)
            cs_v[csl] = cc
            daf_v[csl] = da
            dpf_v[csl] = dp

        @pl.loop(0, S * (TCH // 16))
        def _pb(i):
            s = lax.shift_right_logical(i, 3)
            u = i & 7
            t = 16 * u + lane
            acc = cs_v[pl.ds(s * 4 * TCH + 16 * u, 16)]
            for k in range(1, K):
                acc = acc + cs_v[pl.ds((s * 4 + k) * TCH + 16 * u, 16)]
            off = offs_v[pl.ds(c * TCH + 16 * u, 16)]
            svec = jnp.full((16,), 0, jnp.int32) + s
            gb = plsc.load_gather(blk_b, [svec, t, off])
            logit = gb + gs * (acc * jnp.float32(0.25))
            gate_v[pl.ds(s * TCH + 16 * u, 16)] = jnp.float32(1.0) / (
                jnp.float32(1.0) + jnp.exp(-logit))

        @pl.loop(0, 12 * (TCH // 16))
        def _pc(i):
            p = lax.shift_right_logical(i, 3)
            u = i & 7
            s = lax.shift_right_logical(p, 2)
            k = p & 3
            st_base = s * (BPW * K) + c * 512 + k * 128
            gsl = pl.ds(s * TCH + 16 * u, 16)
            ssl = pl.ds(st_base + 16 * u, 16)
            csl = pl.ds(p * TCH + 16 * u, 16)
            g = gate_v[gsl]
            omg = jnp.float32(1.0) - g
            sa_v[ssl] = g * sa_v[ssl] + omg * daf_v[csl]
            sp_v[ssl] = g * sp_v[ssl] + omg * dpf_v[csl]

    for s in range(S):
        src = pl.ds(s * BPW * K, BPW * K)
        dst = pl.ds(s * (B * K) + wid * BPW * K, BPW * K)
        pltpu.sync_copy(sa_v.at[src], na_hbm.at[dst])
        pltpu.sync_copy(sp_v.at[src], np_hbm.at[dst])


def _compiler_params():
    cp = pltpu.CompilerParams()
    fields = pltpu.CompilerParams.__dataclass_fields__
    if "needs_layout_passes" in fields:
        cp = dataclasses.replace(cp, needs_layout_passes=False)
    if "use_tc_tiling_on_sc" in fields:
        cp = dataclasses.replace(cp, use_tc_tiling_on_sc=False)
    return cp


def _sc_call(ampg, phaseg, biasg, ids1d, sa_lin, sp_lin, gs16):
    f32 = jnp.float32
    i32 = jnp.int32
    run = pl.kernel(
        _sc_body,
        compiler_params=_compiler_params(),
        out_type=(jax.ShapeDtypeStruct((B * S * K,), f32),
                  jax.ShapeDtypeStruct((B * S * K,), f32)),
        mesh=plsc.VectorSubcoreMesh(core_axis_name="c", subcore_axis_name="s"),
        scratch_types=[
            pltpu.VMEM((BPW,), i32),
            pltpu.VMEM((BPW,), i32),
            pltpu.VMEM((NCH, 15, TCH), i32),
            pltpu.VMEM((12, TCH, 16), f32),
            pltpu.VMEM((12, TCH, 16), f32),
            pltpu.VMEM((S, TCH, 16), f32),
            pltpu.VMEM((BPW * S * K,), f32),
            pltpu.VMEM((BPW * S * K,), f32),
            pltpu.VMEM((12 * TCH,), f32),
            pltpu.VMEM((12 * TCH,), f32),
            pltpu.VMEM((12 * TCH,), f32),
            pltpu.VMEM((S * TCH,), f32),
            pltpu.VMEM((16,), f32),
            pltpu.SemaphoreType.DMA,
            pltpu.SemaphoreType.DMA,
        ],
    )
    return run(ampg, phaseg, biasg, ids1d, sa_lin, sp_lin, gs16)


@jax.jit
def _gse(ids, state_amp, state_phase, delta_amp, delta_phase, gate_bias,
         gate_sensitivity):
    t_amp = jnp.transpose(delta_amp, (1, 2, 0))
    t_phase = jnp.transpose(delta_phase, (1, 2, 0))
    t_bias = jnp.transpose(gate_bias, (1, 0))[None]
    t_sa = jnp.transpose(state_amp, (1, 2, 0))
    t_sp = jnp.transpose(state_phase, (1, 2, 0))

    amp_lin, phase_lin = _detile_pair(t_amp, t_phase, K, 3, G, NB)
    bias_lin = _detile_one(t_bias, S, 1, GB, NBB)
    sa_lin, sp_lin = _detile_pair(t_sa, t_sp, K, 3, GS, NBS)

    ampg = amp_lin.reshape(DROWS * 8, 16)
    phaseg = phase_lin.reshape(DROWS * 8, 16)
    biasg = bias_lin.reshape(BROWS * 8, 16)
    ids1d = ids.astype(jnp.int32).reshape(B)
    gs16 = jnp.broadcast_to(
        gate_sensitivity.astype(jnp.float32).reshape(()), (16,))

    na_lin, np_lin = _sc_call(ampg, phaseg, biasg, ids1d,
                              sa_lin.reshape(B * S * K),
                              sp_lin.reshape(B * S * K), gs16)

    def back(x):
        x = x.reshape(S, NTS, K, 128)
        x = jnp.transpose(x, (1, 3, 0, 2))
        return x.reshape(B, S, K)

    return back(na_lin), back(np_lin)


def kernel(ids, state_amp, state_phase, delta_amp, delta_phase, gate_bias,
           gate_sensitivity):
    return _gse(ids, state_amp, state_phase, delta_amp, delta_phase,
                gate_bias, gate_sensitivity)


# speedup vs baseline: 64.8228x; 1.0376x over previous
import dataclasses

import jax
import jax.numpy as jnp
from jax import lax
from jax.experimental import pallas as pl
from jax.experimental.pallas import tpu as pltpu
from jax.experimental.pallas import tpu_sc as plsc

B = 16384
S = 3
K = 4
V = 1000000
NC, NS = 2, 16
NW = NC * NS
BPW = B // NW
TCH = 128
NCH = BPW // TCH

G = 1024
GB = 128
GS = 128
NT = -(-V // 128)
NB = -(-NT // G)
NTP = NB * G
NBB = -(-NT // GB)
NTPB = NBB * GB
SPL = NTP * 512
SGR = SPL // 16
DROWS = 3 * NTP * 4
BROWS = NTPB * 3

NTS = B // 128
NBS = NTS // GS
SROWS = 3 * NTS * 4


def _cos(x):
    t = x * jnp.float32(0.15915494309189535)
    half = jnp.where(t >= 0, jnp.float32(0.5), jnp.float32(-0.5))
    n = (t + half).astype(jnp.int32).astype(jnp.float32)
    r = x - n * jnp.float32(6.283185307179586)
    r2 = r * r
    p = jnp.float32(-1.1470745597729725e-11)
    p = p * r2 + jnp.float32(2.08767569878681e-09)
    p = p * r2 + jnp.float32(-2.755731922398589e-07)
    p = p * r2 + jnp.float32(2.48015873015873e-05)
    p = p * r2 + jnp.float32(-0.001388888888888889)
    p = p * r2 + jnp.float32(0.041666666666666664)
    p = p * r2 + jnp.float32(-0.5)
    return p * r2 + jnp.float32(1.0)


def _detile_body(t_ref, o_ref):
    x = t_ref[...]
    sd = x.shape[1]
    parts = [x[0, :, 128 * g:128 * (g + 1)] for g in range(x.shape[2] // 128)]
    o_ref[...] = jnp.concatenate(parts, axis=0)


def _detile2_body(a_ref, b_ref, oa_ref, ob_ref):
    for src, dst in ((a_ref, oa_ref), (b_ref, ob_ref)):
        x = src[...]
        parts = [x[0, :, 128 * g:128 * (g + 1)]
                 for g in range(x.shape[2] // 128)]
        dst[...] = jnp.concatenate(parts, axis=0)


def _detile_pair(a, b, sd, n_sblk, g, nblk):
    rpb = sd * g
    spec_i = pl.BlockSpec((1, sd, 128 * g), lambda s, i: (s, 0, i))
    spec_o = pl.BlockSpec((rpb, 128), lambda s, i: (s * nblk + i, 0))
    rows = n_sblk * nblk * rpb
    return pl.pallas_call(
        _detile2_body,
        grid=(n_sblk, nblk),
        in_specs=[spec_i, spec_i],
        out_specs=[spec_o, spec_o],
        out_shape=[jax.ShapeDtypeStruct((rows, 128), jnp.float32)] * 2,
    )(a, b)


def _detile_one(a, sd, n_sblk, g, nblk):
    rpb = sd * g
    rows = n_sblk * nblk * rpb
    return pl.pallas_call(
        _detile_body,
        grid=(n_sblk, nblk),
        in_specs=[pl.BlockSpec((1, sd, 128 * g), lambda s, i: (s, 0, i))],
        out_specs=pl.BlockSpec((rpb, 128), lambda s, i: (s * nblk + i, 0)),
        out_shape=jax.ShapeDtypeStruct((rows, 128), jnp.float32),
    )(a)


def _sc_body(ampg, phaseg, biasg, ids_hbm, sa_hbm, sp_hbm, gs_hbm,
             na_hbm, np_hbm,
             ids_v, offs_v, idx3, blk_a, blk_p, blk_b,
             sa_v, sp_v, cs_v, daf_v, dpf_v, gate_v, gs_v, sem_g, sem_s):
    wid = lax.axis_index("c") * NS + lax.axis_index("s")
    tok0 = wid * BPW

    pltpu.sync_copy(ids_hbm.at[pl.ds(tok0, BPW)], ids_v)
    st = []
    for s in range(S):
        src = pl.ds(s * (B * K) + wid * BPW * K, BPW * K)
        dst = pl.ds(s * BPW * K, BPW * K)
        st.append(pltpu.async_copy(sa_hbm.at[src], sa_v.at[dst], sem_s))
        st.append(pltpu.async_copy(sp_hbm.at[src], sp_v.at[dst], sem_s))
    pltpu.sync_copy(gs_hbm, gs_v)

    lane = lax.iota(jnp.int32, 16)

    @pl.loop(0, BPW // 16)
    def _mkidx(u):
        c = lax.shift_right_logical(u, 3)
        lg = u & 7
        idv = ids_v[pl.ds(16 * u, 16)]
        vt = lax.shift_right_logical(idv, 7)
        sub = lax.shift_right_logical(idv, 4) & 7
        base_a = vt * 32 + sub
        base_b = vt * 24 + sub
        offs_v[pl.ds(16 * u, 16)] = idv & 15
        sl = pl.ds(16 * lg, 16)
        for p in range(12):
            idx3[c, p, sl] = base_a + ((p // 4) * SGR + (p % 4) * 8)
        for s in range(S):
            idx3[c, 12 + s, sl] = base_b + s * 8

    gs = gs_v[pl.ds(0, 16)]
    for cp in st:
        cp.wait()

    @pl.loop(0, NCH)
    def _chunk(c):
        @pl.loop(0, 12)
        def _fire_d(p):
            pltpu.async_copy(ampg.at[idx3.at[c, p]], blk_a.at[p], sem_g)
            pltpu.async_copy(phaseg.at[idx3.at[c, p]], blk_p.at[p], sem_g)

        @pl.loop(0, S)
        def _fire_b(s):
            pltpu.async_copy(biasg.at[idx3.at[c, 12 + s]], blk_b.at[s], sem_g)

        @pl.loop(0, 27)
        def _drain(i):
            pltpu.make_async_copy(
                ampg.at[pl.ds(0, TCH)], blk_a.at[0], sem_g).wait()

        @pl.loop(0, 12 * (TCH // 16))
        def _pa(i):
            p = lax.shift_right_logical(i, 3)
            u = i & 7
            s = lax.shift_right_logical(p, 2)
            k = p & 3
            st_base = s * (BPW * K) + c * 512 + k * 128
            t = 16 * u + lane
            off = offs_v[pl.ds(c * TCH + 16 * u, 16)]
            pvec = jnp.full((16,), 0, jnp.int32) + p
            da = plsc.load_gather(blk_a, [pvec, t, off])
            dp = plsc.load_gather(blk_p, [pvec, t, off])
            ssl = pl.ds(st_base + 16 * u, 16)
            csl = pl.ds(p * TCH + 16 * u, 16)
            cc = sa_v[ssl] * da * _cos(sp_v[ssl] - dp)
            cs_v[csl] = cc
            daf_v[csl] = da
            dpf_v[csl] = dp

        @pl.loop(0, S * (TCH // 16))
        def _pb(i):
            s = lax.shift_right_logical(i, 3)
            u = i & 7
            t = 16 * u + lane
            acc = cs_v[pl.ds(s * 4 * TCH + 16 * u, 16)]
            for k in range(1, K):
                acc = acc + cs_v[pl.ds((s * 4 + k) * TCH + 16 * u, 16)]
            off = offs_v[pl.ds(c * TCH + 16 * u, 16)]
            svec = jnp.full((16,), 0, jnp.int32) + s
            gb = plsc.load_gather(blk_b, [svec, t, off])
            logit = gb + gs * (acc * jnp.float32(0.25))
            gate_v[pl.ds(s * TCH + 16 * u, 16)] = jnp.float32(1.0) / (
                jnp.float32(1.0) + jnp.exp(-logit))

        @pl.loop(0, 12 * (TCH // 16))
        def _pc(i):
            p = lax.shift_right_logical(i, 3)
            u = i & 7
            s = lax.shift_right_logical(p, 2)
            k = p & 3
            st_base = s * (BPW * K) + c * 512 + k * 128
            gsl = pl.ds(s * TCH + 16 * u, 16)
            ssl = pl.ds(st_base + 16 * u, 16)
            csl = pl.ds(p * TCH + 16 * u, 16)
            g = gate_v[gsl]
            omg = jnp.float32(1.0) - g
            sa_v[ssl] = g * sa_v[ssl] + omg * daf_v[csl]
            sp_v[ssl] = g * sp_v[ssl] + omg * dpf_v[csl]

    for s in range(S):
        src = pl.ds(s * BPW * K, BPW * K)
        dst = pl.ds(s * (B * K) + wid * BPW * K, BPW * K)
        pltpu.sync_copy(sa_v.at[src], na_hbm.at[dst])
        pltpu.sync_copy(sp_v.at[src], np_hbm.at[dst])


def _compiler_params():
    cp = pltpu.CompilerParams()
    fields = pltpu.CompilerParams.__dataclass_fields__
    if "needs_layout_passes" in fields:
        cp = dataclasses.replace(cp, needs_layout_passes=False)
    if "use_tc_tiling_on_sc" in fields:
        cp = dataclasses.replace(cp, use_tc_tiling_on_sc=False)
    return cp


def _sc_call(ampg, phaseg, biasg, ids1d, sa_lin, sp_lin, gs16):
    f32 = jnp.float32
    i32 = jnp.int32
    run = pl.kernel(
        _sc_body,
        compiler_params=_compiler_params(),
        out_type=(jax.ShapeDtypeStruct((B * S * K,), f32),
                  jax.ShapeDtypeStruct((B * S * K,), f32)),
        mesh=plsc.VectorSubcoreMesh(core_axis_name="c", subcore_axis_name="s"),
        scratch_types=[
            pltpu.VMEM((BPW,), i32),
            pltpu.VMEM((BPW,), i32),
            pltpu.VMEM((NCH, 15, TCH), i32),
            pltpu.VMEM((12, TCH, 16), f32),
            pltpu.VMEM((12, TCH, 16), f32),
            pltpu.VMEM((S, TCH, 16), f32),
            pltpu.VMEM((BPW * S * K,), f32),
            pltpu.VMEM((BPW * S * K,), f32),
            pltpu.VMEM((12 * TCH,), f32),
            pltpu.VMEM((12 * TCH,), f32),
            pltpu.VMEM((12 * TCH,), f32),
            pltpu.VMEM((S * TCH,), f32),
            pltpu.VMEM((16,), f32),
            pltpu.SemaphoreType.DMA,
            pltpu.SemaphoreType.DMA,
        ],
    )
    return run(ampg, phaseg, biasg, ids1d, sa_lin, sp_lin, gs16)


@jax.jit
def _gse(ids, state_amp, state_phase, delta_amp, delta_phase, gate_bias,
         gate_sensitivity):
    t_amp = jnp.transpose(delta_amp, (1, 2, 0))
    t_phase = jnp.transpose(delta_phase, (1, 2, 0))
    t_bias = jnp.transpose(gate_bias, (1, 0))[None]
    t_sa = jnp.transpose(state_amp, (1, 2, 0))
    t_sp = jnp.transpose(state_phase, (1, 2, 0))

    amp_lin, phase_lin = _detile_pair(t_amp, t_phase, K, 3, G, NB)
    bias_lin = _detile_one(t_bias, S, 1, GB, NBB)
    sa_lin, sp_lin = _detile_pair(t_sa, t_sp, K, 3, GS, NBS)

    ampg = amp_lin.reshape(DROWS * 8, 16)
    phaseg = phase_lin.reshape(DROWS * 8, 16)
    biasg = bias_lin.reshape(BROWS * 8, 16)
    ids1d = ids.astype(jnp.int32).reshape(B)
    gs16 = jnp.broadcast_to(
        gate_sensitivity.astype(jnp.float32).reshape(()), (16,))

    na_lin, np_lin = _sc_call(ampg, phaseg, biasg, ids1d,
                              sa_lin.reshape(B * S * K),
                              sp_lin.reshape(B * S * K), gs16)

    def back(x):
        x = x.reshape(S, NTS, K, 128)
        x = jnp.transpose(x, (1, 3, 0, 2))
        return x.reshape(B, S, K)

    return back(na_lin), back(np_lin)


def kernel(ids, state_amp, state_phase, delta_amp, delta_phase, gate_bias,
           gate_sensitivity):
    return _gse(ids, state_amp, state_phase, delta_amp, delta_phase,
                gate_bias, gate_sensitivity)


# speedup vs baseline: 74.1437x; 1.1438x over previous
import dataclasses

import jax
import jax.numpy as jnp
from jax import lax
from jax.experimental import pallas as pl
from jax.experimental.pallas import tpu as pltpu
from jax.experimental.pallas import tpu_sc as plsc

B = 16384
S = 3
K = 4
V = 1000000
NC, NS = 2, 16
NW = NC * NS
BPW = B // NW
TCH = 128
NCH = BPW // TCH

G = 1024
GB = 512
GS = 128
NT = -(-V // 128)
NB = -(-NT // G)
NTP = NB * G
NBB = -(-NT // GB)
NTPB = NBB * GB
SPL = NTP * 512
SGR = SPL // 16
DROWS = 3 * NTP * 4
BROWS = NTPB * 3

NTS = B // 128
NBS = NTS // GS
SROWS = 3 * NTS * 4


def _cos(x):
    t = x * jnp.float32(0.15915494309189535)
    half = jnp.where(t >= 0, jnp.float32(0.5), jnp.float32(-0.5))
    n = (t + half).astype(jnp.int32).astype(jnp.float32)
    r = x - n * jnp.float32(6.283185307179586)
    r2 = r * r
    p = jnp.float32(-1.1470745597729725e-11)
    p = p * r2 + jnp.float32(2.08767569878681e-09)
    p = p * r2 + jnp.float32(-2.755731922398589e-07)
    p = p * r2 + jnp.float32(2.48015873015873e-05)
    p = p * r2 + jnp.float32(-0.001388888888888889)
    p = p * r2 + jnp.float32(0.041666666666666664)
    p = p * r2 + jnp.float32(-0.5)
    return p * r2 + jnp.float32(1.0)


def _detile_body(t_ref, o_ref):
    x = t_ref[...]
    sd = x.shape[1]
    parts = [x[0, :, 128 * g:128 * (g + 1)] for g in range(x.shape[2] // 128)]
    o_ref[...] = jnp.concatenate(parts, axis=0)


def _detile2_body(a_ref, b_ref, oa_ref, ob_ref):
    for src, dst in ((a_ref, oa_ref), (b_ref, ob_ref)):
        x = src[...]
        parts = [x[0, :, 128 * g:128 * (g + 1)]
                 for g in range(x.shape[2] // 128)]
        dst[...] = jnp.concatenate(parts, axis=0)


def _detile_pair(a, b, sd, n_sblk, g, nblk):
    rpb = sd * g
    spec_i = pl.BlockSpec((1, sd, 128 * g), lambda s, i: (s, 0, i))
    spec_o = pl.BlockSpec((rpb, 128), lambda s, i: (s * nblk + i, 0))
    rows = n_sblk * nblk * rpb
    return pl.pallas_call(
        _detile2_body,
        grid=(n_sblk, nblk),
        in_specs=[spec_i, spec_i],
        out_specs=[spec_o, spec_o],
        out_shape=[jax.ShapeDtypeStruct((rows, 128), jnp.float32)] * 2,
    )(a, b)


def _detile_one(a, sd, n_sblk, g, nblk):
    rpb = sd * g
    rows = n_sblk * nblk * rpb
    return pl.pallas_call(
        _detile_body,
        grid=(n_sblk, nblk),
        in_specs=[pl.BlockSpec((1, sd, 128 * g), lambda s, i: (s, 0, i))],
        out_specs=pl.BlockSpec((rpb, 128), lambda s, i: (s * nblk + i, 0)),
        out_shape=jax.ShapeDtypeStruct((rows, 128), jnp.float32),
    )(a)


def _sc_body(ampg, phaseg, biasg, ids_hbm, sa_hbm, sp_hbm, gs_hbm,
             na_hbm, np_hbm,
             ids_v, offs_v, idx3, blk_a, blk_p, blk_b,
             sa_v, sp_v, cs_v, daf_v, dpf_v, gate_v, gs_v, sem_g, sem_s):
    wid = lax.axis_index("c") * NS + lax.axis_index("s")
    tok0 = wid * BPW

    pltpu.sync_copy(ids_hbm.at[pl.ds(tok0, BPW)], ids_v)
    st = []
    for s in range(S):
        src = pl.ds(s * (B * K) + wid * BPW * K, BPW * K)
        dst = pl.ds(s * BPW * K, BPW * K)
        st.append(pltpu.async_copy(sa_hbm.at[src], sa_v.at[dst], sem_s))
        st.append(pltpu.async_copy(sp_hbm.at[src], sp_v.at[dst], sem_s))
    pltpu.sync_copy(gs_hbm, gs_v)

    lane = lax.iota(jnp.int32, 16)

    @pl.loop(0, BPW // 16)
    def _mkidx(u):
        c = lax.shift_right_logical(u, 3)
        lg = u & 7
        idv = ids_v[pl.ds(16 * u, 16)]
        vt = lax.shift_right_logical(idv, 7)
        sub = lax.shift_right_logical(idv, 4) & 7
        base_a = vt * 32 + sub
        base_b = vt * 24 + sub
        offs_v[pl.ds(16 * u, 16)] = idv & 15
        sl = pl.ds(16 * lg, 16)
        for p in range(12):
            idx3[c, p, sl] = base_a + ((p // 4) * SGR + (p % 4) * 8)
        for s in range(S):
            idx3[c, 12 + s, sl] = base_b + s * 8

    gs = gs_v[pl.ds(0, 16)]
    for cp in st:
        cp.wait()

    @pl.loop(0, NCH)
    def _chunk(c):
        @pl.loop(0, 12)
        def _fire_d(p):
            pltpu.async_copy(ampg.at[idx3.at[c, p]], blk_a.at[p], sem_g)
            pltpu.async_copy(phaseg.at[idx3.at[c, p]], blk_p.at[p], sem_g)

        @pl.loop(0, S)
        def _fire_b(s):
            pltpu.async_copy(biasg.at[idx3.at[c, 12 + s]], blk_b.at[s], sem_g)

        @pl.loop(0, 27)
        def _drain(i):
            pltpu.make_async_copy(
                ampg.at[pl.ds(0, TCH)], blk_a.at[0], sem_g).wait()

        @pl.loop(0, 12 * (TCH // 16))
        def _pa(i):
            p = lax.shift_right_logical(i, 3)
            u = i & 7
            s = lax.shift_right_logical(p, 2)
            k = p & 3
            st_base = s * (BPW * K) + c * 512 + k * 128
            t = 16 * u + lane
            off = offs_v[pl.ds(c * TCH + 16 * u, 16)]
            pvec = jnp.full((16,), 0, jnp.int32) + p
            da = plsc.load_gather(blk_a, [pvec, t, off])
            dp = plsc.load_gather(blk_p, [pvec, t, off])
            ssl = pl.ds(st_base + 16 * u, 16)
            csl = pl.ds(p * TCH + 16 * u, 16)
            cc = sa_v[ssl] * da * _cos(sp_v[ssl] - dp)
            cs_v[csl] = cc
            daf_v[csl] = da
            dpf_v[csl] = dp

        @pl.loop(0, S * (TCH // 16))
        def _pb(i):
            s = lax.shift_right_logical(i, 3)
            u = i & 7
            t = 16 * u + lane
            acc = cs_v[pl.ds(s * 4 * TCH + 16 * u, 16)]
            for k in range(1, K):
                acc = acc + cs_v[pl.ds((s * 4 + k) * TCH + 16 * u, 16)]
            off = offs_v[pl.ds(c * TCH + 16 * u, 16)]
            svec = jnp.full((16,), 0, jnp.int32) + s
            gb = plsc.load_gather(blk_b, [svec, t, off])
            logit = gb + gs * (acc * jnp.float32(0.25))
            gate_v[pl.ds(s * TCH + 16 * u, 16)] = jnp.float32(1.0) / (
                jnp.float32(1.0) + jnp.exp(-logit))

        @pl.loop(0, 12 * (TCH // 16))
        def _pc(i):
            p = lax.shift_right_logical(i, 3)
            u = i & 7
            s = lax.shift_right_logical(p, 2)
            k = p & 3
            st_base = s * (BPW * K) + c * 512 + k * 128
            gsl = pl.ds(s * TCH + 16 * u, 16)
            ssl = pl.ds(st_base + 16 * u, 16)
            csl = pl.ds(p * TCH + 16 * u, 16)
            g = gate_v[gsl]
            omg = jnp.float32(1.0) - g
            sa_v[ssl] = g * sa_v[ssl] + omg * daf_v[csl]
            sp_v[ssl] = g * sp_v[ssl] + omg * dpf_v[csl]

    for s in range(S):
        src = pl.ds(s * BPW * K, BPW * K)
        dst = pl.ds(s * (B * K) + wid * BPW * K, BPW * K)
        pltpu.sync_copy(sa_v.at[src], na_hbm.at[dst])
        pltpu.sync_copy(sp_v.at[src], np_hbm.at[dst])


def _compiler_params():
    cp = pltpu.CompilerParams()
    fields = pltpu.CompilerParams.__dataclass_fields__
    if "needs_layout_passes" in fields:
        cp = dataclasses.replace(cp, needs_layout_passes=False)
    if "use_tc_tiling_on_sc" in fields:
        cp = dataclasses.replace(cp, use_tc_tiling_on_sc=False)
    return cp


def _sc_call(ampg, phaseg, biasg, ids1d, sa_lin, sp_lin, gs16):
    f32 = jnp.float32
    i32 = jnp.int32
    run = pl.kernel(
        _sc_body,
        compiler_params=_compiler_params(),
        out_type=(jax.ShapeDtypeStruct((B * S * K,), f32),
                  jax.ShapeDtypeStruct((B * S * K,), f32)),
        mesh=plsc.VectorSubcoreMesh(core_axis_name="c", subcore_axis_name="s"),
        scratch_types=[
            pltpu.VMEM((BPW,), i32),
            pltpu.VMEM((BPW,), i32),
            pltpu.VMEM((NCH, 15, TCH), i32),
            pltpu.VMEM((12, TCH, 16), f32),
            pltpu.VMEM((12, TCH, 16), f32),
            pltpu.VMEM((S, TCH, 16), f32),
            pltpu.VMEM((BPW * S * K,), f32),
            pltpu.VMEM((BPW * S * K,), f32),
            pltpu.VMEM((12 * TCH,), f32),
            pltpu.VMEM((12 * TCH,), f32),
            pltpu.VMEM((12 * TCH,), f32),
            pltpu.VMEM((S * TCH,), f32),
            pltpu.VMEM((16,), f32),
            pltpu.SemaphoreType.DMA,
            pltpu.SemaphoreType.DMA,
        ],
    )
    return run(ampg, phaseg, biasg, ids1d, sa_lin, sp_lin, gs16)


@jax.jit
def _gse(ids, state_amp, state_phase, delta_amp, delta_phase, gate_bias,
         gate_sensitivity):
    t_amp = jnp.transpose(delta_amp, (1, 2, 0))
    t_phase = jnp.transpose(delta_phase, (1, 2, 0))
    t_bias = jnp.transpose(gate_bias, (1, 0))[None]
    t_sa = jnp.transpose(state_amp, (1, 2, 0))
    t_sp = jnp.transpose(state_phase, (1, 2, 0))

    amp_lin, phase_lin = _detile_pair(t_amp, t_phase, K, 3, G, NB)
    bias_lin = _detile_one(t_bias, S, 1, GB, NBB)
    sa_lin, sp_lin = _detile_pair(t_sa, t_sp, K, 3, GS, NBS)

    ampg = amp_lin.reshape(DROWS * 8, 16)
    phaseg = phase_lin.reshape(DROWS * 8, 16)
    biasg = bias_lin.reshape(BROWS * 8, 16)
    ids1d = ids.astype(jnp.int32).reshape(B)
    gs16 = jnp.broadcast_to(
        gate_sensitivity.astype(jnp.float32).reshape(()), (16,))

    na_lin, np_lin = _sc_call(ampg, phaseg, biasg, ids1d,
                              sa_lin.reshape(B * S * K),
                              sp_lin.reshape(B * S * K), gs16)

    def back(x):
        x = x.reshape(S, NTS, K, 128)
        x = jnp.transpose(x, (1, 3, 0, 2))
        return x.reshape(B, S, K)

    return back(na_lin), back(np_lin)


def kernel(ids, state_amp, state_phase, delta_amp, delta_phase, gate_bias,
           gate_sensitivity):
    return _gse(ids, state_amp, state_phase, delta_amp, delta_phase,
                gate_bias, gate_sensitivity)


# speedup vs baseline: 81.0929x; 1.0937x over previous
import dataclasses

import jax
import jax.numpy as jnp
from jax import lax
from jax.experimental import pallas as pl
from jax.experimental.pallas import tpu as pltpu
from jax.experimental.pallas import tpu_sc as plsc

B = 16384
S = 3
K = 4
V = 1000000
NC, NS = 2, 16
NW = NC * NS
BPW = B // NW
TCH = 64
NCH = BPW // TCH

G = 1024
GB = 512
GS = 128
NT = -(-V // 128)
NB = -(-NT // G)
NTP = NB * G
NBB = -(-NT // GB)
NTPB = NBB * GB
SPL = NTP * 512
SGR = SPL // 16
DROWS = 3 * NTP * 4
BROWS = NTPB * 3

NTS = B // 128
NBS = NTS // GS
SROWS = 3 * NTS * 4


def _cos(x):
    t = x * jnp.float32(0.15915494309189535)
    half = jnp.where(t >= 0, jnp.float32(0.5), jnp.float32(-0.5))
    n = (t + half).astype(jnp.int32).astype(jnp.float32)
    r = x - n * jnp.float32(6.283185307179586)
    r2 = r * r
    p = jnp.float32(-1.1470745597729725e-11)
    p = p * r2 + jnp.float32(2.08767569878681e-09)
    p = p * r2 + jnp.float32(-2.755731922398589e-07)
    p = p * r2 + jnp.float32(2.48015873015873e-05)
    p = p * r2 + jnp.float32(-0.001388888888888889)
    p = p * r2 + jnp.float32(0.041666666666666664)
    p = p * r2 + jnp.float32(-0.5)
    return p * r2 + jnp.float32(1.0)


def _detile_body(t_ref, o_ref):
    x = t_ref[...]
    sd = x.shape[1]
    parts = [x[0, :, 128 * g:128 * (g + 1)] for g in range(x.shape[2] // 128)]
    o_ref[...] = jnp.concatenate(parts, axis=0)


def _detile2_body(a_ref, b_ref, oa_ref, ob_ref):
    for src, dst in ((a_ref, oa_ref), (b_ref, ob_ref)):
        x = src[...]
        parts = [x[0, :, 128 * g:128 * (g + 1)]
                 for g in range(x.shape[2] // 128)]
        dst[...] = jnp.concatenate(parts, axis=0)


def _detile_pair(a, b, sd, n_sblk, g, nblk):
    rpb = sd * g
    spec_i = pl.BlockSpec((1, sd, 128 * g), lambda s, i: (s, 0, i))
    spec_o = pl.BlockSpec((rpb, 128), lambda s, i: (s * nblk + i, 0))
    rows = n_sblk * nblk * rpb
    return pl.pallas_call(
        _detile2_body,
        grid=(n_sblk, nblk),
        in_specs=[spec_i, spec_i],
        out_specs=[spec_o, spec_o],
        out_shape=[jax.ShapeDtypeStruct((rows, 128), jnp.float32)] * 2,
    )(a, b)


def _detile_one(a, sd, n_sblk, g, nblk):
    rpb = sd * g
    rows = n_sblk * nblk * rpb
    return pl.pallas_call(
        _detile_body,
        grid=(n_sblk, nblk),
        in_specs=[pl.BlockSpec((1, sd, 128 * g), lambda s, i: (s, 0, i))],
        out_specs=pl.BlockSpec((rpb, 128), lambda s, i: (s * nblk + i, 0)),
        out_shape=jax.ShapeDtypeStruct((rows, 128), jnp.float32),
    )(a)


def _sc_body(ampg, phaseg, biasg, ids_hbm, sa_hbm, sp_hbm, gs_hbm,
             na_hbm, np_hbm,
             ids_v, offs_v, idx3, blk_a0, blk_p0, blk_b0,
             blk_a1, blk_p1, blk_b1,
             sa_v, sp_v, cs_v, daf_v, dpf_v, gate_v, gs_v,
             sem_a, sem_b, sem_s):
    wid = lax.axis_index("c") * NS + lax.axis_index("s")
    tok0 = wid * BPW

    pltpu.sync_copy(ids_hbm.at[pl.ds(tok0, BPW)], ids_v)
    st = []
    for s in range(S):
        src = pl.ds(s * (B * K) + wid * BPW * K, BPW * K)
        dst = pl.ds(s * BPW * K, BPW * K)
        st.append(pltpu.async_copy(sa_hbm.at[src], sa_v.at[dst], sem_s))
        st.append(pltpu.async_copy(sp_hbm.at[src], sp_v.at[dst], sem_s))
    pltpu.sync_copy(gs_hbm, gs_v)

    lane = lax.iota(jnp.int32, 16)
    upc = TCH // 16

    @pl.loop(0, BPW // 16)
    def _mkidx(u):
        c = u // upc
        lg = u - c * upc
        idv = ids_v[pl.ds(16 * u, 16)]
        vt = lax.shift_right_logical(idv, 7)
        sub = lax.shift_right_logical(idv, 4) & 7
        base_a = vt * 32 + sub
        base_b = vt * 24 + sub
        offs_v[pl.ds(16 * u, 16)] = idv & 15
        sl = pl.ds(16 * lg, 16)
        for p in range(12):
            idx3[c, p, sl] = base_a + ((p // 4) * SGR + (p % 4) * 8)
        for s in range(S):
            idx3[c, 12 + s, sl] = base_b + s * 8

    gs = gs_v[pl.ds(0, 16)]
    for cp in st:
        cp.wait()

    def fire(c, blk_a, blk_p, blk_b, sem):
        @pl.loop(0, 12)
        def _fire_d(p):
            pltpu.async_copy(ampg.at[idx3.at[c, p]], blk_a.at[p], sem)
            pltpu.async_copy(phaseg.at[idx3.at[c, p]], blk_p.at[p], sem)

        @pl.loop(0, S)
        def _fire_b(s):
            pltpu.async_copy(biasg.at[idx3.at[c, 12 + s]], blk_b.at[s], sem)

    def drain(blk_a, sem):
        @pl.loop(0, 27)
        def _drain(i):
            pltpu.make_async_copy(
                ampg.at[pl.ds(0, TCH)], blk_a.at[0], sem).wait()

    def compute(c, blk_a, blk_p, blk_b):
        @pl.loop(0, 12 * upc)
        def _pa(i):
            p = i // upc
            u = i - p * upc
            s = lax.shift_right_logical(p, 2)
            k = p & 3
            st_base = (s * (BPW * K) + lax.shift_right_logical(c, 1) * 512
                       + k * 128 + (c & 1) * TCH)
            t = 16 * u + lane
            off = offs_v[pl.ds(c * TCH + 16 * u, 16)]
            pvec = jnp.full((16,), 0, jnp.int32) + p
            da = plsc.load_gather(blk_a, [pvec, t, off])
            dp = plsc.load_gather(blk_p, [pvec, t, off])
            ssl = pl.ds(st_base + 16 * u, 16)
            csl = pl.ds(p * TCH + 16 * u, 16)
            cc = sa_v[ssl] * da * _cos(sp_v[ssl] - dp)
            cs_v[csl] = cc
            daf_v[csl] = da
            dpf_v[csl] = dp

        @pl.loop(0, S * upc)
        def _pb(i):
            s = i // upc
            u = i - s * upc
            t = 16 * u + lane
            acc = cs_v[pl.ds(s * 4 * TCH + 16 * u, 16)]
            for k in range(1, K):
                acc = acc + cs_v[pl.ds((s * 4 + k) * TCH + 16 * u, 16)]
            off = offs_v[pl.ds(c * TCH + 16 * u, 16)]
            svec = jnp.full((16,), 0, jnp.int32) + s
            gb = plsc.load_gather(blk_b, [svec, t, off])
            logit = gb + gs * (acc * jnp.float32(0.25))
            gate_v[pl.ds(s * TCH + 16 * u, 16)] = jnp.float32(1.0) / (
                jnp.float32(1.0) + jnp.exp(-logit))

        @pl.loop(0, 12 * upc)
        def _pc(i):
            p = i // upc
            u = i - p * upc
            s = lax.shift_right_logical(p, 2)
            k = p & 3
            st_base = (s * (BPW * K) + lax.shift_right_logical(c, 1) * 512
                       + k * 128 + (c & 1) * TCH)
            gsl = pl.ds(s * TCH + 16 * u, 16)
            ssl = pl.ds(st_base + 16 * u, 16)
            csl = pl.ds(p * TCH + 16 * u, 16)
            g = gate_v[gsl]
            omg = jnp.float32(1.0) - g
            sa_v[ssl] = g * sa_v[ssl] + omg * daf_v[csl]
            sp_v[ssl] = g * sp_v[ssl] + omg * dpf_v[csl]

    fire(jnp.int32(0), blk_a0, blk_p0, blk_b0, sem_a)

    @pl.loop(0, NCH // 2)
    def _pair(j):
        c0 = 2 * j
        c1 = 2 * j + 1
        fire(c1, blk_a1, blk_p1, blk_b1, sem_b)
        drain(blk_a0, sem_a)
        compute(c0, blk_a0, blk_p0, blk_b0)

        @pl.when(j < NCH // 2 - 1)
        def _pf():
            fire(c0 + 2, blk_a0, blk_p0, blk_b0, sem_a)

        drain(blk_a1, sem_b)
        compute(c1, blk_a1, blk_p1, blk_b1)

    for s in range(S):
        src = pl.ds(s * BPW * K, BPW * K)
        dst = pl.ds(s * (B * K) + wid * BPW * K, BPW * K)
        pltpu.sync_copy(sa_v.at[src], na_hbm.at[dst])
        pltpu.sync_copy(sp_v.at[src], np_hbm.at[dst])


def _compiler_params():
    cp = pltpu.CompilerParams()
    fields = pltpu.CompilerParams.__dataclass_fields__
    if "needs_layout_passes" in fields:
        cp = dataclasses.replace(cp, needs_layout_passes=False)
    if "use_tc_tiling_on_sc" in fields:
        cp = dataclasses.replace(cp, use_tc_tiling_on_sc=False)
    return cp


def _sc_call(ampg, phaseg, biasg, ids1d, sa_lin, sp_lin, gs16):
    f32 = jnp.float32
    i32 = jnp.int32
    run = pl.kernel(
        _sc_body,
        compiler_params=_compiler_params(),
        out_type=(jax.ShapeDtypeStruct((B * S * K,), f32),
                  jax.ShapeDtypeStruct((B * S * K,), f32)),
        mesh=plsc.VectorSubcoreMesh(core_axis_name="c", subcore_axis_name="s"),
        scratch_types=[
            pltpu.VMEM((BPW,), i32),
            pltpu.VMEM((BPW,), i32),
            pltpu.VMEM((NCH, 15, TCH), i32),
            pltpu.VMEM((12, TCH, 16), f32),
            pltpu.VMEM((12, TCH, 16), f32),
            pltpu.VMEM((S, TCH, 16), f32),
            pltpu.VMEM((12, TCH, 16), f32),
            pltpu.VMEM((12, TCH, 16), f32),
            pltpu.VMEM((S, TCH, 16), f32),
            pltpu.VMEM((BPW * S * K,), f32),
            pltpu.VMEM((BPW * S * K,), f32),
            pltpu.VMEM((12 * TCH,), f32),
            pltpu.VMEM((12 * TCH,), f32),
            pltpu.VMEM((12 * TCH,), f32),
            pltpu.VMEM((S * TCH,), f32),
            pltpu.VMEM((16,), f32),
            pltpu.SemaphoreType.DMA,
            pltpu.SemaphoreType.DMA,
            pltpu.SemaphoreType.DMA,
        ],
    )
    return run(ampg, phaseg, biasg, ids1d, sa_lin, sp_lin, gs16)


@jax.jit
def _gse(ids, state_amp, state_phase, delta_amp, delta_phase, gate_bias,
         gate_sensitivity):
    t_amp = jnp.transpose(delta_amp, (1, 2, 0))
    t_phase = jnp.transpose(delta_phase, (1, 2, 0))
    t_bias = jnp.transpose(gate_bias, (1, 0))[None]
    t_sa = jnp.transpose(state_amp, (1, 2, 0))
    t_sp = jnp.transpose(state_phase, (1, 2, 0))

    amp_lin, phase_lin = _detile_pair(t_amp, t_phase, K, 3, G, NB)
    bias_lin = _detile_one(t_bias, S, 1, GB, NBB)
    sa_lin, sp_lin = _detile_pair(t_sa, t_sp, K, 3, GS, NBS)

    ampg = amp_lin.reshape(DROWS * 8, 16)
    phaseg = phase_lin.reshape(DROWS * 8, 16)
    biasg = bias_lin.reshape(BROWS * 8, 16)
    ids1d = ids.astype(jnp.int32).reshape(B)
    gs16 = jnp.broadcast_to(
        gate_sensitivity.astype(jnp.float32).reshape(()), (16,))

    na_lin, np_lin = _sc_call(ampg, phaseg, biasg, ids1d,
                              sa_lin.reshape(B * S * K),
                              sp_lin.reshape(B * S * K), gs16)

    def back(x):
        x = x.reshape(S, NTS, K, 128)
        x = jnp.transpose(x, (1, 3, 0, 2))
        return x.reshape(B, S, K)

    return back(na_lin), back(np_lin)


def kernel(ids, state_amp, state_phase, delta_amp, delta_phase, gate_bias,
           gate_sensitivity):
    return _gse(ids, state_amp, state_phase, delta_amp, delta_phase,
                gate_bias, gate_sensitivity)
